# baseline (speedup 1.0000x reference)
import functools

import jax
import jax.numpy as jnp
from jax import lax
from jax.experimental import pallas as pl
from jax.experimental.pallas import tpu as pltpu
from jax.experimental.pallas import tpu_sc as plsc

_NC = 2
_NS = 16

_E_WORKERS = 25
_E_CHUNK = 400
_E_CHUNKS = 16
_X_WORKERS = 5
_X_CHUNK = 200
_X_CHUNKS = 10


@functools.partial(
    pl.kernel,
    out_type=(
        jax.ShapeDtypeStruct((10000, 256), jnp.float32),
        jax.ShapeDtypeStruct((160000, 16), jnp.float32),
    ),
    mesh=plsc.VectorSubcoreMesh(core_axis_name="c", subcore_axis_name="s"),
    scratch_types=[
        pltpu.VMEM((_X_CHUNK, 256), jnp.float32),
        pltpu.VMEM((_E_CHUNK, 16), jnp.float32),
    ],
    compiler_params=pltpu.CompilerParams(use_tc_tiling_on_sc=True),
)
def _copy_all(x_hbm, e_hbm, xo_hbm, eo_hbm, xbuf, ebuf):
    wid = lax.axis_index("s") * _NC + lax.axis_index("c")

    @pl.when(wid < _E_WORKERS)
    def _():
        for j in range(_E_CHUNKS):
            base = wid * (_E_CHUNK * _E_CHUNKS) + j * _E_CHUNK
            pltpu.sync_copy(e_hbm.at[pl.ds(base, _E_CHUNK)], ebuf)
            pltpu.sync_copy(ebuf, eo_hbm.at[pl.ds(base, _E_CHUNK)])

    @pl.when((wid >= _E_WORKERS) & (wid < _E_WORKERS + _X_WORKERS))
    def _():
        for j in range(_X_CHUNKS):
            base = (wid - _E_WORKERS) * (_X_CHUNK * _X_CHUNKS) + j * _X_CHUNK
            pltpu.sync_copy(x_hbm.at[pl.ds(base, _X_CHUNK)], xbuf)
            pltpu.sync_copy(xbuf, xo_hbm.at[pl.ds(base, _X_CHUNK)])


def kernel(x, edge_index, edge_attr):
    del edge_index
    return tuple(_copy_all(x, edge_attr))


# speedup vs baseline: 1.2509x; 1.2509x over previous
import jax
import jax.numpy as jnp
from jax.experimental import pallas as pl
from jax.experimental.pallas import tpu as pltpu

_NBUF = 4
_ECHUNK = 8000
_NCHUNK = 160000 // _ECHUNK


def _copy_body(x_hbm, e_hbm, xo_hbm, eo_hbm,
               xbuf, eb0, eb1, eb2, eb3,
               sx_in, sx_out, si0, si1, si2, si3, so0, so1, so2, so3):
    ebufs = (eb0, eb1, eb2, eb3)
    sin = (si0, si1, si2, si3)
    sout = (so0, so1, so2, so3)

    def e_in(i):
        return pltpu.make_async_copy(
            e_hbm.at[pl.ds(i * _ECHUNK, _ECHUNK)], ebufs[i % _NBUF], sin[i % _NBUF])

    def e_out(i):
        return pltpu.make_async_copy(
            ebufs[i % _NBUF], eo_hbm.at[pl.ds(i * _ECHUNK, _ECHUNK)], sout[i % _NBUF])

    x_in = pltpu.make_async_copy(x_hbm, xbuf, sx_in)
    x_in.start()
    for i in range(_NBUF):
        e_in(i).start()
    x_in.wait()
    x_out = pltpu.make_async_copy(xbuf, xo_hbm, sx_out)
    x_out.start()
    for i in range(_NCHUNK):
        e_in(i).wait()
        e_out(i).start()
        j = i + _NBUF
        if j < _NCHUNK:
            e_out(j - _NBUF).wait()
            e_in(j).start()
    for i in range(_NCHUNK - _NBUF, _NCHUNK):
        e_out(i).wait()
    x_out.wait()


def kernel(x, edge_index, edge_attr):
    del edge_index
    x_out, e_out = pl.pallas_call(
        _copy_body,
        in_specs=[
            pl.BlockSpec(memory_space=pl.ANY),
            pl.BlockSpec(memory_space=pl.ANY),
        ],
        out_specs=[
            pl.BlockSpec(memory_space=pl.ANY),
            pl.BlockSpec(memory_space=pl.ANY),
        ],
        out_shape=[
            jax.ShapeDtypeStruct(x.shape, x.dtype),
            jax.ShapeDtypeStruct(edge_attr.shape, edge_attr.dtype),
        ],
        scratch_shapes=[
            pltpu.VMEM((10000, 256), jnp.float32),
            pltpu.VMEM((_ECHUNK, 16), jnp.float32),
            pltpu.VMEM((_ECHUNK, 16), jnp.float32),
            pltpu.VMEM((_ECHUNK, 16), jnp.float32),
            pltpu.VMEM((_ECHUNK, 16), jnp.float32),
        ] + [pltpu.SemaphoreType.DMA] * 10,
    )(x, edge_attr)
    return (x_out, e_out)
